# Initial kernel scaffold; baseline (speedup 1.0000x reference)
import jax
import jax.numpy as jnp
from jax.experimental import pallas as pl


def kernel(x, edge_index, Wl0, Wr0, b0, Wl1, Wr1, b1, Wl2, Wr2, b2):
    raise NotImplementedError("write your pallas kernel here")


# speedup vs baseline: 5.2855x; 5.2855x over previous
import functools

import jax
import jax.numpy as jnp
from jax import lax
from jax.experimental import pallas as pl
from jax.experimental.pallas import tpu as pltpu
from jax.experimental.pallas import tpu_sc as plsc

NC = 2
NS = 16
BATCH = 125


def _make_sc_aggregate(N, E, D):
    HALF = D // NC
    NB = E // (NS * BATCH)
    RPT = (N // NS) // 8 * 8
    RLAST = N - RPT * (NS - 1)

    mesh = plsc.VectorSubcoreMesh(core_axis_name="c", subcore_axis_name="s")

    @functools.partial(
        pl.kernel,
        out_type=(
            jax.ShapeDtypeStruct((NC, N, HALF), jnp.float32),
            jax.ShapeDtypeStruct((NC, N), jnp.float32),
        ),
        mesh=mesh,
        scratch_types=[
            pltpu.VMEM((NB, BATCH), jnp.int32),
            pltpu.VMEM((NB, BATCH), jnp.int32),
            pltpu.VMEM((BATCH, HALF), jnp.float32),
            pltpu.VMEM((BATCH,), jnp.float32),
            pltpu.VMEM_SHARED((N, HALF), jnp.float32),
            pltpu.VMEM_SHARED((N,), jnp.float32),
            pltpu.SemaphoreType.DMA,
        ],
    )
    def agg(h_hbm, edges_hbm, zrow_hbm, zcnt_hbm, ones_hbm,
            sum_out, cnt_out,
            src_v, dst_v, rows_v, ones_v, acc, cnt_acc, sem):
        c = lax.axis_index("c")
        s = lax.axis_index("s")

        pltpu.sync_copy(edges_hbm.at[0].at[s], src_v)
        pltpu.sync_copy(edges_hbm.at[1].at[s], dst_v)
        pltpu.sync_copy(ones_hbm, ones_v)

        @pl.when(s < NS - 1)
        def _():
            pltpu.sync_copy(zrow_hbm.at[pl.ds(0, RPT)],
                            acc.at[pl.ds(s * RPT, RPT)])

        @pl.when(s == NS - 1)
        def _():
            pltpu.sync_copy(zrow_hbm, acc.at[pl.ds((NS - 1) * RPT, RLAST)])

        @pl.when(s == 0)
        def _():
            pltpu.sync_copy(zcnt_hbm, cnt_acc)

        plsc.subcore_barrier()

        def body(j, carry):
            pltpu.async_copy(h_hbm.at[c].at[src_v.at[j]], rows_v,
                             sem).wait()
            pltpu.sync_copy(rows_v, acc.at[dst_v.at[j]], add=True)
            pltpu.sync_copy(ones_v, cnt_acc.at[dst_v.at[j]], add=True)
            return carry

        lax.fori_loop(0, NB, body, 0)

        plsc.subcore_barrier()

        @pl.when(s < NS - 1)
        def _():
            pltpu.sync_copy(acc.at[pl.ds(s * RPT, RPT)],
                            sum_out.at[c].at[pl.ds(s * RPT, RPT)])

        @pl.when(s == NS - 1)
        def _():
            pltpu.sync_copy(acc.at[pl.ds((NS - 1) * RPT, RLAST)],
                            sum_out.at[c].at[pl.ds((NS - 1) * RPT, RLAST)])

        @pl.when(s == 0)
        def _():
            pltpu.sync_copy(cnt_acc, cnt_out.at[c])

    return agg


def _make_tc_dense(N, D, final):
    HALF = D // 2
    BLK = 1000
    grid = (N // BLK,)

    def body(sum_ref, cnt_ref, h_ref, wl_ref, wr_ref, b_ref, o_ref):
        scale = 1.0 / jnp.maximum(cnt_ref[...], 1.0)
        m0 = sum_ref[0] * scale
        m1 = sum_ref[1] * scale
        z = (
            jnp.dot(m0, wl_ref[:HALF, :], preferred_element_type=jnp.float32)
            + jnp.dot(m1, wl_ref[HALF:, :], preferred_element_type=jnp.float32)
            + jnp.dot(h_ref[0], wr_ref[:HALF, :], preferred_element_type=jnp.float32)
            + jnp.dot(h_ref[1], wr_ref[HALF:, :], preferred_element_type=jnp.float32)
            + b_ref[...]
        )
        z = jnp.maximum(z, 0.0)
        if final:
            o_ref[...] = z
        else:
            o_ref[0] = z[:, :HALF]
            o_ref[1] = z[:, HALF:]

    split_spec = pl.BlockSpec((2, BLK, HALF), lambda i: (0, i, 0))
    if final:
        out_shape = jax.ShapeDtypeStruct((N, D), jnp.float32)
        out_spec = pl.BlockSpec((BLK, D), lambda i: (i, 0))
    else:
        out_shape = jax.ShapeDtypeStruct((2, N, HALF), jnp.float32)
        out_spec = split_spec

    return pl.pallas_call(
        body,
        grid=grid,
        in_specs=[
            split_spec,
            pl.BlockSpec((BLK, 1), lambda i: (i, 0)),
            split_spec,
            pl.BlockSpec((D, D), lambda i: (0, 0)),
            pl.BlockSpec((D, D), lambda i: (0, 0)),
            pl.BlockSpec((1, D), lambda i: (0, 0)),
        ],
        out_specs=out_spec,
        out_shape=out_shape,
    )


def kernel(x, edge_index, Wl0, Wr0, b0, Wl1, Wr1, b1, Wl2, Wr2, b2):
    N, D = x.shape
    E = edge_index.shape[1]
    HALF = D // NC
    NB = E // (NS * BATCH)

    agg = _make_sc_aggregate(N, E, D)
    dense_mid = _make_tc_dense(N, D, final=False)
    dense_fin = _make_tc_dense(N, D, final=True)

    edges_r = edge_index.reshape(2, NS, NB, BATCH)
    zrow = jnp.zeros((N - (N // NS) // 8 * 8 * (NS - 1), HALF), jnp.float32)
    zcnt = jnp.zeros((N,), jnp.float32)
    ones = jnp.ones((BATCH,), jnp.float32)

    h = jnp.stack([x[:, :HALF], x[:, HALF:]])
    layers = ((Wl0, Wr0, b0), (Wl1, Wr1, b1), (Wl2, Wr2, b2))
    for i, (Wl, Wr, b) in enumerate(layers):
        summed, cnt = agg(h, edges_r, zrow, zcnt, ones)
        cnt0 = cnt[0].reshape(N, 1)
        b2d = b.reshape(1, D)
        if i < 2:
            h = dense_mid(summed, cnt0, h, Wl, Wr, b2d)
        else:
            h = dense_fin(summed, cnt0, h, Wl, Wr, b2d)
    return h


# speedup vs baseline: 6.9256x; 1.3103x over previous
import functools

import jax
import jax.numpy as jnp
from jax import lax
from jax.experimental import pallas as pl
from jax.experimental.pallas import tpu as pltpu
from jax.experimental.pallas import tpu_sc as plsc

NC = 2
NS = 16
BATCH = 125
CH = 16


def _make_sc_aggregate(N, E, D, with_counts):
    HALF = D // NC
    NB = E // (NS * BATCH)
    NCHUNK = NB // CH
    RPT = (N // NS) // 8 * 8
    RLAST = N - RPT * (NS - 1)

    mesh = plsc.VectorSubcoreMesh(core_axis_name="c", subcore_axis_name="s")

    out_type = [jax.ShapeDtypeStruct((NC, N, HALF), jnp.float32)]
    if with_counts:
        out_type.append(jax.ShapeDtypeStruct((NC, N), jnp.float32))

    @functools.partial(
        pl.kernel,
        out_type=tuple(out_type),
        mesh=mesh,
        scratch_types=[
            pltpu.VMEM((2, CH, BATCH), jnp.int32),
            pltpu.VMEM((2, CH, BATCH), jnp.int32),
            pltpu.VMEM((2, BATCH, HALF), jnp.float32),
            pltpu.VMEM((BATCH,), jnp.float32),
            pltpu.VMEM_SHARED((N, HALF), jnp.float32),
            pltpu.VMEM_SHARED((N,), jnp.float32),
            pltpu.SemaphoreType.DMA,
            pltpu.SemaphoreType.DMA,
            pltpu.SemaphoreType.DMA,
            pltpu.SemaphoreType.DMA,
        ],
    )
    def agg(h_hbm, edges_hbm, zrow_hbm, zcnt_hbm, ones_hbm, *refs):
        if with_counts:
            sum_out, cnt_out = refs[0], refs[1]
            refs = refs[2:]
        else:
            sum_out = refs[0]
            refs = refs[1:]
        (src_v, dst_v, rows_v, ones_v, acc, cnt_acc,
         semr0, semr1, semis, semid) = refs
        semr = (semr0, semr1)
        c = lax.axis_index("c")
        s = lax.axis_index("s")

        if with_counts:
            pltpu.sync_copy(ones_hbm, ones_v)

        @pl.when(s < NS - 1)
        def _():
            pltpu.sync_copy(zrow_hbm.at[pl.ds(0, RPT)],
                            acc.at[pl.ds(s * RPT, RPT)])

        @pl.when(s == NS - 1)
        def _():
            pltpu.sync_copy(zrow_hbm, acc.at[pl.ds((NS - 1) * RPT, RLAST)])

        if with_counts:
            @pl.when(s == 0)
            def _():
                pltpu.sync_copy(zcnt_hbm, cnt_acc)

        plsc.subcore_barrier()

        pltpu.sync_copy(edges_hbm.at[0].at[s].at[0], src_v.at[0])
        pltpu.sync_copy(edges_hbm.at[1].at[s].at[0], dst_v.at[0])
        pltpu.async_copy(h_hbm.at[c].at[src_v.at[0].at[0]], rows_v.at[0],
                         semr[0])

        for k in range(NCHUNK):
            ks = k % 2
            kn = (k + 1) % 2
            if k + 1 < NCHUNK:
                pltpu.async_copy(edges_hbm.at[0].at[s].at[k + 1],
                                 src_v.at[kn], semis)
                pltpu.async_copy(edges_hbm.at[1].at[s].at[k + 1],
                                 dst_v.at[kn], semid)

            def body(p, carry, k=k, ks=ks, kn=kn):
                for b in (0, 1):
                    r = 2 * p + b
                    slot = b
                    other = 1 - b
                    pltpu.make_async_copy(
                        h_hbm.at[c].at[src_v.at[ks].at[r]],
                        rows_v.at[slot], semr[slot]).wait()

                    if k + 1 < NCHUNK:
                        @pl.when(r == CH - 1)
                        def _():
                            pltpu.make_async_copy(
                                edges_hbm.at[0].at[s].at[k + 1],
                                src_v.at[kn], semis).wait()
                            pltpu.make_async_copy(
                                edges_hbm.at[1].at[s].at[k + 1],
                                dst_v.at[kn], semid).wait()
                            pltpu.async_copy(
                                h_hbm.at[c].at[src_v.at[kn].at[0]],
                                rows_v.at[other], semr[other])

                    @pl.when(r < CH - 1)
                    def _():
                        pltpu.async_copy(
                            h_hbm.at[c].at[src_v.at[ks].at[r + 1]],
                            rows_v.at[other], semr[other])

                    pltpu.sync_copy(rows_v.at[slot],
                                    acc.at[dst_v.at[ks].at[r]], add=True)
                    if with_counts:
                        pltpu.sync_copy(ones_v,
                                        cnt_acc.at[dst_v.at[ks].at[r]],
                                        add=True)
                return carry

            lax.fori_loop(0, CH // 2, body, 0)

        plsc.subcore_barrier()

        @pl.when(s < NS - 1)
        def _():
            pltpu.sync_copy(acc.at[pl.ds(s * RPT, RPT)],
                            sum_out.at[c].at[pl.ds(s * RPT, RPT)])

        @pl.when(s == NS - 1)
        def _():
            pltpu.sync_copy(acc.at[pl.ds((NS - 1) * RPT, RLAST)],
                            sum_out.at[c].at[pl.ds((NS - 1) * RPT, RLAST)])

        if with_counts:
            @pl.when(s == 0)
            def _():
                pltpu.sync_copy(cnt_acc, cnt_out.at[c])

    return agg


def _make_tc_right(N, D):
    HALF = D // 2
    BLK = 1000

    def body(h_ref, wr_ref, b_ref, o_ref):
        z = (
            jnp.dot(h_ref[0], wr_ref[:HALF, :],
                    preferred_element_type=jnp.float32)
            + jnp.dot(h_ref[1], wr_ref[HALF:, :],
                      preferred_element_type=jnp.float32)
            + b_ref[...]
        )
        o_ref[0] = z[:, :HALF]
        o_ref[1] = z[:, HALF:]

    split_spec = pl.BlockSpec((2, BLK, HALF), lambda i: (0, i, 0))
    return pl.pallas_call(
        body,
        grid=(N // BLK,),
        in_specs=[
            split_spec,
            pl.BlockSpec((D, D), lambda i: (0, 0)),
            pl.BlockSpec((1, D), lambda i: (0, 0)),
        ],
        out_specs=split_spec,
        out_shape=jax.ShapeDtypeStruct((2, N, HALF), jnp.float32),
    )


def _make_tc_combine(N, D, final):
    HALF = D // 2
    BLK = 1000

    def body(sum_ref, cnt_ref, r_ref, wl_ref, o_ref):
        scale = 1.0 / jnp.maximum(cnt_ref[...], 1.0)
        m0 = sum_ref[0] * scale
        m1 = sum_ref[1] * scale
        z = (
            jnp.dot(m0, wl_ref[:HALF, :], preferred_element_type=jnp.float32)
            + jnp.dot(m1, wl_ref[HALF:, :],
                      preferred_element_type=jnp.float32)
        )
        z = z + jnp.concatenate([r_ref[0], r_ref[1]], axis=1)
        z = jnp.maximum(z, 0.0)
        if final:
            o_ref[...] = z
        else:
            o_ref[0] = z[:, :HALF]
            o_ref[1] = z[:, HALF:]

    split_spec = pl.BlockSpec((2, BLK, HALF), lambda i: (0, i, 0))
    if final:
        out_shape = jax.ShapeDtypeStruct((N, D), jnp.float32)
        out_spec = pl.BlockSpec((BLK, D), lambda i: (i, 0))
    else:
        out_shape = jax.ShapeDtypeStruct((2, N, HALF), jnp.float32)
        out_spec = split_spec

    return pl.pallas_call(
        body,
        grid=(N // BLK,),
        in_specs=[
            split_spec,
            pl.BlockSpec((BLK, 1), lambda i: (i, 0)),
            split_spec,
            pl.BlockSpec((D, D), lambda i: (0, 0)),
        ],
        out_specs=out_spec,
        out_shape=out_shape,
    )


def kernel(x, edge_index, Wl0, Wr0, b0, Wl1, Wr1, b1, Wl2, Wr2, b2):
    N, D = x.shape
    E = edge_index.shape[1]
    HALF = D // NC
    NB = E // (NS * BATCH)

    agg0 = _make_sc_aggregate(N, E, D, with_counts=True)
    agg = _make_sc_aggregate(N, E, D, with_counts=False)
    tc_right = _make_tc_right(N, D)
    comb_mid = _make_tc_combine(N, D, final=False)
    comb_fin = _make_tc_combine(N, D, final=True)

    edges_r = edge_index.reshape(2, NS, NB // CH, CH, BATCH)
    zrow = jnp.zeros((N - (N // NS) // 8 * 8 * (NS - 1), HALF), jnp.float32)
    zcnt = jnp.zeros((N,), jnp.float32)
    ones = jnp.ones((BATCH,), jnp.float32)

    h = jnp.stack([x[:, :HALF], x[:, HALF:]])
    cnt0 = None
    layers = ((Wl0, Wr0, b0), (Wl1, Wr1, b1), (Wl2, Wr2, b2))
    for i, (Wl, Wr, b) in enumerate(layers):
        right = tc_right(h, Wr, b.reshape(1, D))
        if i == 0:
            summed, cnt = agg0(h, edges_r, zrow, zcnt, ones)
            cnt0 = cnt[0].reshape(N, 1)
        else:
            (summed,) = agg(h, edges_r, zrow, zcnt, ones)
        if i < 2:
            h = comb_mid(summed, cnt0, right, Wl)
        else:
            h = comb_fin(summed, cnt0, right, Wl)
    return h


# speedup vs baseline: 6.9258x; 1.0000x over previous
import functools

import jax
import jax.numpy as jnp
from jax import lax
from jax.experimental import pallas as pl
from jax.experimental.pallas import tpu as pltpu
from jax.experimental.pallas import tpu_sc as plsc

NC = 2
NS = 16
BATCH = 125
CH = 16


def _make_sc_aggregate(N, E, D, with_counts):
    HALF = D // NC
    NB = E // (NS * BATCH)
    NCHUNK = NB // CH
    RPT = (N // NS) // 8 * 8
    RLAST = N - RPT * (NS - 1)

    mesh = plsc.VectorSubcoreMesh(core_axis_name="c", subcore_axis_name="s")

    out_type = [jax.ShapeDtypeStruct((NC, N, HALF), jnp.float32)]
    if with_counts:
        out_type.append(jax.ShapeDtypeStruct((NC, N), jnp.float32))

    @functools.partial(
        pl.kernel,
        out_type=tuple(out_type),
        mesh=mesh,
        scratch_types=[
            pltpu.VMEM((2, CH, BATCH), jnp.int32),
            pltpu.VMEM((2, CH, BATCH), jnp.int32),
            pltpu.VMEM((2, BATCH, HALF), jnp.float32),
            pltpu.VMEM((BATCH,), jnp.float32),
            pltpu.VMEM_SHARED((N, HALF), jnp.float32),
            pltpu.VMEM_SHARED((N,), jnp.float32),
            pltpu.SemaphoreType.DMA,
            pltpu.SemaphoreType.DMA,
            pltpu.SemaphoreType.DMA,
            pltpu.SemaphoreType.DMA,
            pltpu.SemaphoreType.DMA,
            pltpu.SemaphoreType.DMA,
        ],
    )
    def agg(h_hbm, edges_hbm, zrow_hbm, zcnt_hbm, ones_hbm, *refs):
        if with_counts:
            sum_out, cnt_out = refs[0], refs[1]
            refs = refs[2:]
        else:
            sum_out = refs[0]
            refs = refs[1:]
        (src_v, dst_v, rows_v, ones_v, acc, cnt_acc,
         semr0, semr1, semw0, semw1, semis, semid) = refs
        semr = (semr0, semr1)
        semw = (semw0, semw1)
        c = lax.axis_index("c")
        s = lax.axis_index("s")

        if with_counts:
            pltpu.sync_copy(ones_hbm, ones_v)

        @pl.when(s < NS - 1)
        def _():
            pltpu.sync_copy(zrow_hbm.at[pl.ds(0, RPT)],
                            acc.at[pl.ds(s * RPT, RPT)])

        @pl.when(s == NS - 1)
        def _():
            pltpu.sync_copy(zrow_hbm, acc.at[pl.ds((NS - 1) * RPT, RLAST)])

        if with_counts:
            @pl.when(s == 0)
            def _():
                pltpu.sync_copy(zcnt_hbm, cnt_acc)

        plsc.subcore_barrier()

        pltpu.sync_copy(edges_hbm.at[0].at[s].at[0], src_v.at[0])
        pltpu.sync_copy(edges_hbm.at[1].at[s].at[0], dst_v.at[0])
        pltpu.async_copy(h_hbm.at[c].at[src_v.at[0].at[0]], rows_v.at[0],
                         semr[0])

        for k in range(NCHUNK):
            ks = k % 2
            kn = (k + 1) % 2

            def body(p, carry, k=k, ks=ks, kn=kn):
                for b in (0, 1):
                    r = 2 * p + b
                    slot = b
                    other = 1 - b
                    pltpu.make_async_copy(
                        h_hbm.at[c].at[src_v.at[ks].at[r]],
                        rows_v.at[slot], semr[slot]).wait()

                    def wait_scatter():
                        pltpu.make_async_copy(
                            rows_v.at[other],
                            acc.at[dst_v.at[ks].at[r]], semw[other]).wait()
                        if with_counts:
                            pltpu.make_async_copy(
                                ones_v, cnt_acc.at[dst_v.at[ks].at[r]],
                                semw[other]).wait()

                    if k == 0:
                        @pl.when(r > 0)
                        def _():
                            wait_scatter()
                    else:
                        wait_scatter()

                    if k + 1 < NCHUNK:
                        @pl.when(r == 1)
                        def _():
                            pltpu.async_copy(edges_hbm.at[0].at[s].at[k + 1],
                                             src_v.at[kn], semis)
                            pltpu.async_copy(edges_hbm.at[1].at[s].at[k + 1],
                                             dst_v.at[kn], semid)

                        @pl.when(r == CH - 1)
                        def _():
                            pltpu.make_async_copy(
                                edges_hbm.at[0].at[s].at[k + 1],
                                src_v.at[kn], semis).wait()
                            pltpu.make_async_copy(
                                edges_hbm.at[1].at[s].at[k + 1],
                                dst_v.at[kn], semid).wait()
                            pltpu.async_copy(
                                h_hbm.at[c].at[src_v.at[kn].at[0]],
                                rows_v.at[other], semr[other])

                    @pl.when(r < CH - 1)
                    def _():
                        pltpu.async_copy(
                            h_hbm.at[c].at[src_v.at[ks].at[r + 1]],
                            rows_v.at[other], semr[other])

                    pltpu.async_copy(rows_v.at[slot],
                                     acc.at[dst_v.at[ks].at[r]], semw[slot],
                                     add=True)
                    if with_counts:
                        pltpu.async_copy(ones_v,
                                         cnt_acc.at[dst_v.at[ks].at[r]],
                                         semw[slot], add=True)
                return carry

            lax.fori_loop(0, CH // 2, body, 0)

        pltpu.make_async_copy(rows_v.at[1], acc.at[dst_v.at[(NCHUNK - 1) % 2]
                                                   .at[CH - 1]],
                              semw[1]).wait()
        if with_counts:
            pltpu.make_async_copy(ones_v,
                                  cnt_acc.at[dst_v.at[(NCHUNK - 1) % 2]
                                             .at[CH - 1]],
                                  semw[1]).wait()

        plsc.subcore_barrier()

        @pl.when(s < NS - 1)
        def _():
            pltpu.sync_copy(acc.at[pl.ds(s * RPT, RPT)],
                            sum_out.at[c].at[pl.ds(s * RPT, RPT)])

        @pl.when(s == NS - 1)
        def _():
            pltpu.sync_copy(acc.at[pl.ds((NS - 1) * RPT, RLAST)],
                            sum_out.at[c].at[pl.ds((NS - 1) * RPT, RLAST)])

        if with_counts:
            @pl.when(s == 0)
            def _():
                pltpu.sync_copy(cnt_acc, cnt_out.at[c])

    return agg


def _make_tc_right(N, D):
    HALF = D // 2
    BLK = 1000

    def body(h_ref, wr_ref, b_ref, o_ref):
        z = (
            jnp.dot(h_ref[0], wr_ref[:HALF, :],
                    preferred_element_type=jnp.float32)
            + jnp.dot(h_ref[1], wr_ref[HALF:, :],
                      preferred_element_type=jnp.float32)
            + b_ref[...]
        )
        o_ref[0] = z[:, :HALF]
        o_ref[1] = z[:, HALF:]

    split_spec = pl.BlockSpec((2, BLK, HALF), lambda i: (0, i, 0))
    return pl.pallas_call(
        body,
        grid=(N // BLK,),
        in_specs=[
            split_spec,
            pl.BlockSpec((D, D), lambda i: (0, 0)),
            pl.BlockSpec((1, D), lambda i: (0, 0)),
        ],
        out_specs=split_spec,
        out_shape=jax.ShapeDtypeStruct((2, N, HALF), jnp.float32),
    )


def _make_tc_combine(N, D, final):
    HALF = D // 2
    BLK = 1000

    def body(sum_ref, cnt_ref, r_ref, wl_ref, o_ref):
        scale = 1.0 / jnp.maximum(cnt_ref[...], 1.0)
        m0 = sum_ref[0] * scale
        m1 = sum_ref[1] * scale
        z = (
            jnp.dot(m0, wl_ref[:HALF, :], preferred_element_type=jnp.float32)
            + jnp.dot(m1, wl_ref[HALF:, :],
                      preferred_element_type=jnp.float32)
        )
        z = z + jnp.concatenate([r_ref[0], r_ref[1]], axis=1)
        z = jnp.maximum(z, 0.0)
        if final:
            o_ref[...] = z
        else:
            o_ref[0] = z[:, :HALF]
            o_ref[1] = z[:, HALF:]

    split_spec = pl.BlockSpec((2, BLK, HALF), lambda i: (0, i, 0))
    if final:
        out_shape = jax.ShapeDtypeStruct((N, D), jnp.float32)
        out_spec = pl.BlockSpec((BLK, D), lambda i: (i, 0))
    else:
        out_shape = jax.ShapeDtypeStruct((2, N, HALF), jnp.float32)
        out_spec = split_spec

    return pl.pallas_call(
        body,
        grid=(N // BLK,),
        in_specs=[
            split_spec,
            pl.BlockSpec((BLK, 1), lambda i: (i, 0)),
            split_spec,
            pl.BlockSpec((D, D), lambda i: (0, 0)),
        ],
        out_specs=out_spec,
        out_shape=out_shape,
    )


def kernel(x, edge_index, Wl0, Wr0, b0, Wl1, Wr1, b1, Wl2, Wr2, b2):
    N, D = x.shape
    E = edge_index.shape[1]
    HALF = D // NC
    NB = E // (NS * BATCH)

    agg0 = _make_sc_aggregate(N, E, D, with_counts=True)
    agg = _make_sc_aggregate(N, E, D, with_counts=False)
    tc_right = _make_tc_right(N, D)
    comb_mid = _make_tc_combine(N, D, final=False)
    comb_fin = _make_tc_combine(N, D, final=True)

    edges_r = edge_index.reshape(2, NS, NB // CH, CH, BATCH)
    zrow = jnp.zeros((N - (N // NS) // 8 * 8 * (NS - 1), HALF), jnp.float32)
    zcnt = jnp.zeros((N,), jnp.float32)
    ones = jnp.ones((BATCH,), jnp.float32)

    h = jnp.stack([x[:, :HALF], x[:, HALF:]])
    cnt0 = None
    layers = ((Wl0, Wr0, b0), (Wl1, Wr1, b1), (Wl2, Wr2, b2))
    for i, (Wl, Wr, b) in enumerate(layers):
        right = tc_right(h, Wr, b.reshape(1, D))
        if i == 0:
            summed, cnt = agg0(h, edges_r, zrow, zcnt, ones)
            cnt0 = cnt[0].reshape(N, 1)
        else:
            (summed,) = agg(h, edges_r, zrow, zcnt, ones)
        if i < 2:
            h = comb_mid(summed, cnt0, right, Wl)
        else:
            h = comb_fin(summed, cnt0, right, Wl)
    return h


# speedup vs baseline: 7.9817x; 1.1525x over previous
import functools

import jax
import jax.numpy as jnp
from jax import lax
from jax.experimental import pallas as pl
from jax.experimental.pallas import tpu as pltpu
from jax.experimental.pallas import tpu_sc as plsc

NC = 2
NS = 16
BATCH = 125
CH = 16


def _make_sc_aggregate(N, E, D, with_counts):
    HALF = D // NC
    NB = E // (NS * BATCH)
    NCHUNK = NB // CH
    RPT = (N // NS) // 8 * 8
    RLAST = N - RPT * (NS - 1)

    mesh = plsc.VectorSubcoreMesh(core_axis_name="c", subcore_axis_name="s")

    out_type = [jax.ShapeDtypeStruct((NC, N, HALF), jnp.float32)]
    if with_counts:
        out_type.append(jax.ShapeDtypeStruct((NC, N), jnp.float32))

    @functools.partial(
        pl.kernel,
        out_type=tuple(out_type),
        mesh=mesh,
        scratch_types=[
            pltpu.VMEM((2, CH, BATCH), jnp.int32),
            pltpu.VMEM((2, CH, BATCH), jnp.int32),
            pltpu.VMEM((2, BATCH, HALF), jnp.float32),
            pltpu.VMEM((BATCH,), jnp.float32),
            pltpu.VMEM_SHARED((N, HALF), jnp.float32),
            pltpu.VMEM_SHARED((N,), jnp.float32),
            pltpu.SemaphoreType.DMA,
            pltpu.SemaphoreType.DMA,
            pltpu.SemaphoreType.DMA,
            pltpu.SemaphoreType.DMA,
            pltpu.SemaphoreType.DMA,
            pltpu.SemaphoreType.DMA,
        ],
    )
    def agg(h_hbm, edges_hbm, zrow_hbm, zcnt_hbm, ones_hbm, *refs):
        if with_counts:
            sum_out, cnt_out = refs[0], refs[1]
            refs = refs[2:]
        else:
            sum_out = refs[0]
            refs = refs[1:]
        (src_v, dst_v, rows_v, ones_v, acc, cnt_acc,
         semr0, semr1, semw0, semw1, semis, semid) = refs
        semr = (semr0, semr1)
        semw = (semw0, semw1)
        c = lax.axis_index("c")
        s = lax.axis_index("s")

        if with_counts:
            pltpu.sync_copy(ones_hbm, ones_v)

        @pl.when(s < NS - 1)
        def _():
            pltpu.sync_copy(zrow_hbm.at[pl.ds(0, RPT)],
                            acc.at[pl.ds(s * RPT, RPT)])

        @pl.when(s == NS - 1)
        def _():
            pltpu.sync_copy(zrow_hbm, acc.at[pl.ds((NS - 1) * RPT, RLAST)])

        if with_counts:
            @pl.when(s == 0)
            def _():
                pltpu.sync_copy(zcnt_hbm, cnt_acc)

        plsc.subcore_barrier()

        pltpu.sync_copy(edges_hbm.at[0].at[s].at[0], src_v.at[0])
        pltpu.sync_copy(edges_hbm.at[1].at[s].at[0], dst_v.at[0])
        pltpu.async_copy(h_hbm.at[c].at[src_v.at[0].at[0]], rows_v.at[0],
                         semr[0])

        for k in range(NCHUNK):
            ks = k % 2
            kn = (k + 1) % 2

            def body(p, carry, k=k, ks=ks, kn=kn):
                for b in (0, 1):
                    r = 2 * p + b
                    slot = b
                    other = 1 - b

                    def wait_scatter():
                        pltpu.make_async_copy(
                            rows_v.at[other],
                            acc.at[dst_v.at[ks].at[r]], semw[other]).wait()
                        if with_counts:
                            pltpu.make_async_copy(
                                ones_v, cnt_acc.at[dst_v.at[ks].at[r]],
                                semw[other]).wait()

                    if k == 0:
                        @pl.when(r > 0)
                        def _():
                            wait_scatter()
                    else:
                        wait_scatter()

                    if k + 1 < NCHUNK:
                        @pl.when(r == 1)
                        def _():
                            pltpu.async_copy(edges_hbm.at[0].at[s].at[k + 1],
                                             src_v.at[kn], semis)
                            pltpu.async_copy(edges_hbm.at[1].at[s].at[k + 1],
                                             dst_v.at[kn], semid)

                        @pl.when(r == CH - 1)
                        def _():
                            pltpu.make_async_copy(
                                edges_hbm.at[0].at[s].at[k + 1],
                                src_v.at[kn], semis).wait()
                            pltpu.make_async_copy(
                                edges_hbm.at[1].at[s].at[k + 1],
                                dst_v.at[kn], semid).wait()
                            pltpu.async_copy(
                                h_hbm.at[c].at[src_v.at[kn].at[0]],
                                rows_v.at[other], semr[other])

                    @pl.when(r < CH - 1)
                    def _():
                        pltpu.async_copy(
                            h_hbm.at[c].at[src_v.at[ks].at[r + 1]],
                            rows_v.at[other], semr[other])

                    pltpu.make_async_copy(
                        h_hbm.at[c].at[src_v.at[ks].at[r]],
                        rows_v.at[slot], semr[slot]).wait()

                    pltpu.async_copy(rows_v.at[slot],
                                     acc.at[dst_v.at[ks].at[r]], semw[slot],
                                     add=True)
                    if with_counts:
                        pltpu.async_copy(ones_v,
                                         cnt_acc.at[dst_v.at[ks].at[r]],
                                         semw[slot], add=True)
                return carry

            lax.fori_loop(0, CH // 2, body, 0)

        pltpu.make_async_copy(rows_v.at[1], acc.at[dst_v.at[(NCHUNK - 1) % 2]
                                                   .at[CH - 1]],
                              semw[1]).wait()
        if with_counts:
            pltpu.make_async_copy(ones_v,
                                  cnt_acc.at[dst_v.at[(NCHUNK - 1) % 2]
                                             .at[CH - 1]],
                                  semw[1]).wait()

        plsc.subcore_barrier()

        @pl.when(s < NS - 1)
        def _():
            pltpu.sync_copy(acc.at[pl.ds(s * RPT, RPT)],
                            sum_out.at[c].at[pl.ds(s * RPT, RPT)])

        @pl.when(s == NS - 1)
        def _():
            pltpu.sync_copy(acc.at[pl.ds((NS - 1) * RPT, RLAST)],
                            sum_out.at[c].at[pl.ds((NS - 1) * RPT, RLAST)])

        if with_counts:
            @pl.when(s == 0)
            def _():
                pltpu.sync_copy(cnt_acc, cnt_out.at[c])

    return agg


def _make_tc_right(N, D):
    HALF = D // 2
    BLK = 1000

    def body(h_ref, wr_ref, b_ref, o_ref):
        z = (
            jnp.dot(h_ref[0], wr_ref[:HALF, :],
                    preferred_element_type=jnp.float32)
            + jnp.dot(h_ref[1], wr_ref[HALF:, :],
                      preferred_element_type=jnp.float32)
            + b_ref[...]
        )
        o_ref[0] = z[:, :HALF]
        o_ref[1] = z[:, HALF:]

    split_spec = pl.BlockSpec((2, BLK, HALF), lambda i: (0, i, 0))
    return pl.pallas_call(
        body,
        grid=(N // BLK,),
        in_specs=[
            split_spec,
            pl.BlockSpec((D, D), lambda i: (0, 0)),
            pl.BlockSpec((1, D), lambda i: (0, 0)),
        ],
        out_specs=split_spec,
        out_shape=jax.ShapeDtypeStruct((2, N, HALF), jnp.float32),
    )


def _make_tc_combine(N, D, final):
    HALF = D // 2
    BLK = 1000

    def body(sum_ref, cnt_ref, r_ref, wl_ref, o_ref):
        scale = 1.0 / jnp.maximum(cnt_ref[...], 1.0)
        m0 = sum_ref[0] * scale
        m1 = sum_ref[1] * scale
        z = (
            jnp.dot(m0, wl_ref[:HALF, :], preferred_element_type=jnp.float32)
            + jnp.dot(m1, wl_ref[HALF:, :],
                      preferred_element_type=jnp.float32)
        )
        z = z + jnp.concatenate([r_ref[0], r_ref[1]], axis=1)
        z = jnp.maximum(z, 0.0)
        if final:
            o_ref[...] = z
        else:
            o_ref[0] = z[:, :HALF]
            o_ref[1] = z[:, HALF:]

    split_spec = pl.BlockSpec((2, BLK, HALF), lambda i: (0, i, 0))
    if final:
        out_shape = jax.ShapeDtypeStruct((N, D), jnp.float32)
        out_spec = pl.BlockSpec((BLK, D), lambda i: (i, 0))
    else:
        out_shape = jax.ShapeDtypeStruct((2, N, HALF), jnp.float32)
        out_spec = split_spec

    return pl.pallas_call(
        body,
        grid=(N // BLK,),
        in_specs=[
            split_spec,
            pl.BlockSpec((BLK, 1), lambda i: (i, 0)),
            split_spec,
            pl.BlockSpec((D, D), lambda i: (0, 0)),
        ],
        out_specs=out_spec,
        out_shape=out_shape,
    )


def kernel(x, edge_index, Wl0, Wr0, b0, Wl1, Wr1, b1, Wl2, Wr2, b2):
    N, D = x.shape
    E = edge_index.shape[1]
    HALF = D // NC
    NB = E // (NS * BATCH)

    agg0 = _make_sc_aggregate(N, E, D, with_counts=True)
    agg = _make_sc_aggregate(N, E, D, with_counts=False)
    tc_right = _make_tc_right(N, D)
    comb_mid = _make_tc_combine(N, D, final=False)
    comb_fin = _make_tc_combine(N, D, final=True)

    edges_r = edge_index.reshape(2, NS, NB // CH, CH, BATCH)
    zrow = jnp.zeros((N - (N // NS) // 8 * 8 * (NS - 1), HALF), jnp.float32)
    zcnt = jnp.zeros((N,), jnp.float32)
    ones = jnp.ones((BATCH,), jnp.float32)

    h = jnp.stack([x[:, :HALF], x[:, HALF:]])
    cnt0 = None
    layers = ((Wl0, Wr0, b0), (Wl1, Wr1, b1), (Wl2, Wr2, b2))
    for i, (Wl, Wr, b) in enumerate(layers):
        right = tc_right(h, Wr, b.reshape(1, D))
        if i == 0:
            summed, cnt = agg0(h, edges_r, zrow, zcnt, ones)
            cnt0 = cnt[0].reshape(N, 1)
        else:
            (summed,) = agg(h, edges_r, zrow, zcnt, ones)
        if i < 2:
            h = comb_mid(summed, cnt0, right, Wl)
        else:
            h = comb_fin(summed, cnt0, right, Wl)
    return h


# speedup vs baseline: 8.0735x; 1.0115x over previous
import functools

import jax
import jax.numpy as jnp
from jax import lax
from jax.experimental import pallas as pl
from jax.experimental.pallas import tpu as pltpu
from jax.experimental.pallas import tpu_sc as plsc

NC = 2
NS = 16
BATCH = 125
CH = 16


def _make_sc_aggregate(N, E, D, with_counts):
    HALF = D // NC
    NB = E // (NS * BATCH)
    NCHUNK = NB // CH
    RPT = (N // NS) // 8 * 8
    RLAST = N - RPT * (NS - 1)

    mesh = plsc.VectorSubcoreMesh(core_axis_name="c", subcore_axis_name="s")

    out_type = [jax.ShapeDtypeStruct((NC, N, HALF), jnp.float32)]
    if with_counts:
        out_type.append(jax.ShapeDtypeStruct((NC, N), jnp.float32))

    @functools.partial(
        pl.kernel,
        out_type=tuple(out_type),
        mesh=mesh,
        scratch_types=[
            pltpu.VMEM((2, CH, BATCH), jnp.int32),
            pltpu.VMEM((2, CH, BATCH), jnp.int32),
            pltpu.VMEM((2, BATCH, HALF), jnp.float32),
            pltpu.VMEM((BATCH,), jnp.float32),
            pltpu.VMEM_SHARED((N, HALF), jnp.float32),
            pltpu.VMEM_SHARED((N,), jnp.float32),
            pltpu.SemaphoreType.DMA,
            pltpu.SemaphoreType.DMA,
            pltpu.SemaphoreType.DMA,
            pltpu.SemaphoreType.DMA,
            pltpu.SemaphoreType.DMA,
            pltpu.SemaphoreType.DMA,
        ],
    )
    def agg(h_hbm, edges_hbm, zrow_hbm, zcnt_hbm, ones_hbm, *refs):
        if with_counts:
            sum_out, cnt_out = refs[0], refs[1]
            refs = refs[2:]
        else:
            sum_out = refs[0]
            refs = refs[1:]
        (src_v, dst_v, rows_v, ones_v, acc, cnt_acc,
         semr0, semr1, semw0, semw1, semis, semid) = refs
        semr = (semr0, semr1)
        semw = (semw0, semw1)
        c = lax.axis_index("c")
        s = lax.axis_index("s")

        if with_counts:
            pltpu.sync_copy(ones_hbm, ones_v)

        @pl.when(s < NS - 1)
        def _():
            pltpu.sync_copy(zrow_hbm.at[pl.ds(0, RPT)],
                            acc.at[pl.ds(s * RPT, RPT)])

        @pl.when(s == NS - 1)
        def _():
            pltpu.sync_copy(zrow_hbm, acc.at[pl.ds((NS - 1) * RPT, RLAST)])

        if with_counts:
            @pl.when(s == 0)
            def _():
                pltpu.sync_copy(zcnt_hbm, cnt_acc)

        pltpu.sync_copy(edges_hbm.at[0].at[s].at[0], src_v.at[0])
        pltpu.sync_copy(edges_hbm.at[1].at[s].at[0], dst_v.at[0])
        pltpu.async_copy(h_hbm.at[c].at[src_v.at[0].at[0]], rows_v.at[0],
                         semr[0])

        plsc.subcore_barrier()

        for k in range(NCHUNK):
            ks = k % 2
            kn = (k + 1) % 2

            def body(p, carry, k=k, ks=ks, kn=kn):
                for b in (0, 1):
                    r = 2 * p + b
                    slot = b
                    other = 1 - b

                    def wait_scatter():
                        pltpu.make_async_copy(
                            rows_v.at[other],
                            acc.at[dst_v.at[ks].at[r]], semw[other]).wait()
                        if with_counts:
                            pltpu.make_async_copy(
                                ones_v, cnt_acc.at[dst_v.at[ks].at[r]],
                                semw[other]).wait()

                    if k == 0:
                        @pl.when(r > 0)
                        def _():
                            wait_scatter()
                    else:
                        wait_scatter()

                    if k + 1 < NCHUNK:
                        @pl.when(r == 1)
                        def _():
                            pltpu.async_copy(edges_hbm.at[0].at[s].at[k + 1],
                                             src_v.at[kn], semis)
                            pltpu.async_copy(edges_hbm.at[1].at[s].at[k + 1],
                                             dst_v.at[kn], semid)

                        @pl.when(r == CH - 1)
                        def _():
                            pltpu.make_async_copy(
                                edges_hbm.at[0].at[s].at[k + 1],
                                src_v.at[kn], semis).wait()
                            pltpu.make_async_copy(
                                edges_hbm.at[1].at[s].at[k + 1],
                                dst_v.at[kn], semid).wait()
                            pltpu.async_copy(
                                h_hbm.at[c].at[src_v.at[kn].at[0]],
                                rows_v.at[other], semr[other])

                    @pl.when(r < CH - 1)
                    def _():
                        pltpu.async_copy(
                            h_hbm.at[c].at[src_v.at[ks].at[r + 1]],
                            rows_v.at[other], semr[other])

                    pltpu.make_async_copy(
                        h_hbm.at[c].at[src_v.at[ks].at[r]],
                        rows_v.at[slot], semr[slot]).wait()

                    pltpu.async_copy(rows_v.at[slot],
                                     acc.at[dst_v.at[ks].at[r]], semw[slot],
                                     add=True)
                    if with_counts:
                        pltpu.async_copy(ones_v,
                                         cnt_acc.at[dst_v.at[ks].at[r]],
                                         semw[slot], add=True)
                return carry

            lax.fori_loop(0, CH // 2, body, 0)

        pltpu.make_async_copy(rows_v.at[1], acc.at[dst_v.at[(NCHUNK - 1) % 2]
                                                   .at[CH - 1]],
                              semw[1]).wait()
        if with_counts:
            pltpu.make_async_copy(ones_v,
                                  cnt_acc.at[dst_v.at[(NCHUNK - 1) % 2]
                                             .at[CH - 1]],
                                  semw[1]).wait()

        plsc.subcore_barrier()

        @pl.when(s < NS - 1)
        def _():
            pltpu.sync_copy(acc.at[pl.ds(s * RPT, RPT)],
                            sum_out.at[c].at[pl.ds(s * RPT, RPT)])

        @pl.when(s == NS - 1)
        def _():
            pltpu.sync_copy(acc.at[pl.ds((NS - 1) * RPT, RLAST)],
                            sum_out.at[c].at[pl.ds((NS - 1) * RPT, RLAST)])

        if with_counts:
            @pl.when(s == 0)
            def _():
                pltpu.sync_copy(cnt_acc, cnt_out.at[c])

    return agg


def _make_tc_right(N, D):
    HALF = D // 2
    BLK = 1000

    def body(h_ref, wr_ref, b_ref, o_ref):
        z = (
            jnp.dot(h_ref[0], wr_ref[:HALF, :],
                    preferred_element_type=jnp.float32)
            + jnp.dot(h_ref[1], wr_ref[HALF:, :],
                      preferred_element_type=jnp.float32)
            + b_ref[...]
        )
        o_ref[0] = z[:, :HALF]
        o_ref[1] = z[:, HALF:]

    split_spec = pl.BlockSpec((2, BLK, HALF), lambda i: (0, i, 0))
    return pl.pallas_call(
        body,
        grid=(N // BLK,),
        in_specs=[
            split_spec,
            pl.BlockSpec((D, D), lambda i: (0, 0)),
            pl.BlockSpec((1, D), lambda i: (0, 0)),
        ],
        out_specs=split_spec,
        out_shape=jax.ShapeDtypeStruct((2, N, HALF), jnp.float32),
    )


def _make_tc_combine(N, D, final):
    HALF = D // 2
    BLK = 1000

    def body(sum_ref, cnt_ref, r_ref, wl_ref, o_ref):
        scale = 1.0 / jnp.maximum(cnt_ref[...], 1.0)
        m0 = sum_ref[0] * scale
        m1 = sum_ref[1] * scale
        z = (
            jnp.dot(m0, wl_ref[:HALF, :], preferred_element_type=jnp.float32)
            + jnp.dot(m1, wl_ref[HALF:, :],
                      preferred_element_type=jnp.float32)
        )
        z = z + jnp.concatenate([r_ref[0], r_ref[1]], axis=1)
        z = jnp.maximum(z, 0.0)
        if final:
            o_ref[...] = z
        else:
            o_ref[0] = z[:, :HALF]
            o_ref[1] = z[:, HALF:]

    split_spec = pl.BlockSpec((2, BLK, HALF), lambda i: (0, i, 0))
    if final:
        out_shape = jax.ShapeDtypeStruct((N, D), jnp.float32)
        out_spec = pl.BlockSpec((BLK, D), lambda i: (i, 0))
    else:
        out_shape = jax.ShapeDtypeStruct((2, N, HALF), jnp.float32)
        out_spec = split_spec

    return pl.pallas_call(
        body,
        grid=(N // BLK,),
        in_specs=[
            split_spec,
            pl.BlockSpec((BLK, 1), lambda i: (i, 0)),
            split_spec,
            pl.BlockSpec((D, D), lambda i: (0, 0)),
        ],
        out_specs=out_spec,
        out_shape=out_shape,
    )


def kernel(x, edge_index, Wl0, Wr0, b0, Wl1, Wr1, b1, Wl2, Wr2, b2):
    N, D = x.shape
    E = edge_index.shape[1]
    HALF = D // NC
    NB = E // (NS * BATCH)

    agg0 = _make_sc_aggregate(N, E, D, with_counts=True)
    agg = _make_sc_aggregate(N, E, D, with_counts=False)
    tc_right = _make_tc_right(N, D)
    comb_mid = _make_tc_combine(N, D, final=False)
    comb_fin = _make_tc_combine(N, D, final=True)

    edges_r = edge_index.reshape(2, NS, NB // CH, CH, BATCH)
    zrow = jnp.zeros((N - (N // NS) // 8 * 8 * (NS - 1), HALF), jnp.float32)
    zcnt = jnp.zeros((N,), jnp.float32)
    ones = jnp.ones((BATCH,), jnp.float32)

    h = jnp.stack([x[:, :HALF], x[:, HALF:]])
    cnt0 = None
    layers = ((Wl0, Wr0, b0), (Wl1, Wr1, b1), (Wl2, Wr2, b2))
    for i, (Wl, Wr, b) in enumerate(layers):
        right = tc_right(h, Wr, b.reshape(1, D))
        if i == 0:
            summed, cnt = agg0(h, edges_r, zrow, zcnt, ones)
            cnt0 = cnt[0].reshape(N, 1)
        else:
            (summed,) = agg(h, edges_r, zrow, zcnt, ones)
        if i < 2:
            h = comb_mid(summed, cnt0, right, Wl)
        else:
            h = comb_fin(summed, cnt0, right, Wl)
    return h


# speedup vs baseline: 8.1042x; 1.0038x over previous
import functools

import jax
import jax.numpy as jnp
from jax import lax
from jax.experimental import pallas as pl
from jax.experimental.pallas import tpu as pltpu
from jax.experimental.pallas import tpu_sc as plsc

NC = 2
NS = 16
BATCH = 125
CH = 16


def _make_sc_aggregate(N, E, D, with_counts):
    HALF = D // NC
    NB = E // (NS * BATCH)
    NCHUNK = NB // CH
    RPT = (N // NS) // 8 * 8
    RLAST = N - RPT * (NS - 1)

    mesh = plsc.VectorSubcoreMesh(core_axis_name="c", subcore_axis_name="s")

    out_type = [jax.ShapeDtypeStruct((NC, N, HALF), jnp.float32)]
    if with_counts:
        out_type.append(jax.ShapeDtypeStruct((NC, N), jnp.float32))

    @functools.partial(
        pl.kernel,
        out_type=tuple(out_type),
        mesh=mesh,
        scratch_types=[
            pltpu.VMEM((2, CH, BATCH), jnp.int32),
            pltpu.VMEM((2, CH, BATCH), jnp.int32),
            pltpu.VMEM((2, BATCH, HALF), jnp.float32),
            pltpu.VMEM((BATCH,), jnp.float32),
            pltpu.VMEM_SHARED((N, HALF), jnp.float32),
            pltpu.VMEM_SHARED((N,), jnp.float32),
            pltpu.SemaphoreType.DMA,
            pltpu.SemaphoreType.DMA,
            pltpu.SemaphoreType.DMA,
            pltpu.SemaphoreType.DMA,
            pltpu.SemaphoreType.DMA,
            pltpu.SemaphoreType.DMA,
        ],
    )
    def agg(h_hbm, edges_hbm, zrow_hbm, zcnt_hbm, ones_hbm, *refs):
        if with_counts:
            sum_out, cnt_out = refs[0], refs[1]
            refs = refs[2:]
        else:
            sum_out = refs[0]
            refs = refs[1:]
        (src_v, dst_v, rows_v, ones_v, acc, cnt_acc,
         semr0, semr1, semw0, semw1, semis, semid) = refs
        semr = (semr0, semr1)
        semw = (semw0, semw1)
        c = lax.axis_index("c")
        s = lax.axis_index("s")

        if with_counts:
            pltpu.sync_copy(ones_hbm, ones_v)

        @pl.when(s < NS - 1)
        def _():
            pltpu.sync_copy(zrow_hbm.at[pl.ds(0, RPT)],
                            acc.at[pl.ds(s * RPT, RPT)])

        @pl.when(s == NS - 1)
        def _():
            pltpu.sync_copy(zrow_hbm, acc.at[pl.ds((NS - 1) * RPT, RLAST)])

        if with_counts:
            @pl.when(s == 0)
            def _():
                pltpu.sync_copy(zcnt_hbm, cnt_acc)

        pltpu.sync_copy(edges_hbm.at[0].at[s].at[0], src_v.at[0])
        pltpu.sync_copy(edges_hbm.at[1].at[s].at[0], dst_v.at[0])
        pltpu.async_copy(h_hbm.at[c].at[src_v.at[0].at[0]], rows_v.at[0],
                         semr[0])

        plsc.subcore_barrier()

        for k in range(NCHUNK):
            ks = k % 2
            kn = (k + 1) % 2

            def body(p, carry, k=k, ks=ks, kn=kn):
                for b in (0, 1):
                    r = 2 * p + b
                    slot = b
                    other = 1 - b

                    def wait_scatter():
                        pltpu.make_async_copy(
                            rows_v.at[other],
                            acc.at[dst_v.at[ks].at[r]], semw[other]).wait()
                        if with_counts:
                            pltpu.make_async_copy(
                                ones_v, cnt_acc.at[dst_v.at[ks].at[r]],
                                semw[other]).wait()

                    if k == 0:
                        @pl.when(r > 0)
                        def _():
                            wait_scatter()
                    else:
                        wait_scatter()

                    if k + 1 < NCHUNK:
                        @pl.when(r == 1)
                        def _():
                            pltpu.async_copy(edges_hbm.at[0].at[s].at[k + 1],
                                             src_v.at[kn], semis)
                            pltpu.async_copy(edges_hbm.at[1].at[s].at[k + 1],
                                             dst_v.at[kn], semid)

                        @pl.when(r == CH - 1)
                        def _():
                            pltpu.make_async_copy(
                                edges_hbm.at[0].at[s].at[k + 1],
                                src_v.at[kn], semis).wait()
                            pltpu.make_async_copy(
                                edges_hbm.at[1].at[s].at[k + 1],
                                dst_v.at[kn], semid).wait()
                            pltpu.async_copy(
                                h_hbm.at[c].at[src_v.at[kn].at[0]],
                                rows_v.at[other], semr[other])

                    @pl.when(r < CH - 1)
                    def _():
                        pltpu.async_copy(
                            h_hbm.at[c].at[src_v.at[ks].at[r + 1]],
                            rows_v.at[other], semr[other])

                    pltpu.make_async_copy(
                        h_hbm.at[c].at[src_v.at[ks].at[r]],
                        rows_v.at[slot], semr[slot]).wait()

                    pltpu.async_copy(rows_v.at[slot],
                                     acc.at[dst_v.at[ks].at[r]], semw[slot],
                                     add=True)
                    if with_counts:
                        pltpu.async_copy(ones_v,
                                         cnt_acc.at[dst_v.at[ks].at[r]],
                                         semw[slot], add=True)
                return carry

            lax.fori_loop(0, CH // 2, body, 0)

        pltpu.make_async_copy(rows_v.at[1], acc.at[dst_v.at[(NCHUNK - 1) % 2]
                                                   .at[CH - 1]],
                              semw[1]).wait()
        if with_counts:
            pltpu.make_async_copy(ones_v,
                                  cnt_acc.at[dst_v.at[(NCHUNK - 1) % 2]
                                             .at[CH - 1]],
                                  semw[1]).wait()

        plsc.subcore_barrier()

        @pl.when(s < NS - 1)
        def _():
            pltpu.sync_copy(acc.at[pl.ds(s * RPT, RPT)],
                            sum_out.at[c].at[pl.ds(s * RPT, RPT)])

        @pl.when(s == NS - 1)
        def _():
            pltpu.sync_copy(acc.at[pl.ds((NS - 1) * RPT, RLAST)],
                            sum_out.at[c].at[pl.ds((NS - 1) * RPT, RLAST)])

        if with_counts:
            @pl.when(s == 0)
            def _():
                pltpu.sync_copy(cnt_acc, cnt_out.at[c])

    return agg


def _make_tc_right(N, D):
    HALF = D // 2
    BLK = 1000

    def body(h_ref, wr_ref, b_ref, o_ref):
        h = jnp.concatenate([h_ref[0], h_ref[1]], axis=1)
        z = jnp.dot(h, wr_ref[...],
                    preferred_element_type=jnp.float32) + b_ref[...]
        o_ref[0] = z[:, :HALF]
        o_ref[1] = z[:, HALF:]

    split_spec = pl.BlockSpec((2, BLK, HALF), lambda i: (0, i, 0))
    return pl.pallas_call(
        body,
        grid=(N // BLK,),
        in_specs=[
            split_spec,
            pl.BlockSpec((D, D), lambda i: (0, 0)),
            pl.BlockSpec((1, D), lambda i: (0, 0)),
        ],
        out_specs=split_spec,
        out_shape=jax.ShapeDtypeStruct((2, N, HALF), jnp.float32),
    )


def _make_tc_combine(N, D, final):
    HALF = D // 2
    BLK = 1000

    def body(sum_ref, cnt_ref, r_ref, wl_ref, o_ref):
        scale = 1.0 / jnp.maximum(cnt_ref[...], 1.0)
        m = jnp.concatenate([sum_ref[0], sum_ref[1]], axis=1) * scale
        z = jnp.dot(m, wl_ref[...], preferred_element_type=jnp.float32)
        z = z + jnp.concatenate([r_ref[0], r_ref[1]], axis=1)
        z = jnp.maximum(z, 0.0)
        if final:
            o_ref[...] = z
        else:
            o_ref[0] = z[:, :HALF]
            o_ref[1] = z[:, HALF:]

    split_spec = pl.BlockSpec((2, BLK, HALF), lambda i: (0, i, 0))
    if final:
        out_shape = jax.ShapeDtypeStruct((N, D), jnp.float32)
        out_spec = pl.BlockSpec((BLK, D), lambda i: (i, 0))
    else:
        out_shape = jax.ShapeDtypeStruct((2, N, HALF), jnp.float32)
        out_spec = split_spec

    return pl.pallas_call(
        body,
        grid=(N // BLK,),
        in_specs=[
            split_spec,
            pl.BlockSpec((BLK, 1), lambda i: (i, 0)),
            split_spec,
            pl.BlockSpec((D, D), lambda i: (0, 0)),
        ],
        out_specs=out_spec,
        out_shape=out_shape,
    )


def kernel(x, edge_index, Wl0, Wr0, b0, Wl1, Wr1, b1, Wl2, Wr2, b2):
    N, D = x.shape
    E = edge_index.shape[1]
    HALF = D // NC
    NB = E // (NS * BATCH)

    agg0 = _make_sc_aggregate(N, E, D, with_counts=True)
    agg = _make_sc_aggregate(N, E, D, with_counts=False)
    tc_right = _make_tc_right(N, D)
    comb_mid = _make_tc_combine(N, D, final=False)
    comb_fin = _make_tc_combine(N, D, final=True)

    edges_r = edge_index.reshape(2, NS, NB // CH, CH, BATCH)
    zrow = jnp.zeros((N - (N // NS) // 8 * 8 * (NS - 1), HALF), jnp.float32)
    zcnt = jnp.zeros((N,), jnp.float32)
    ones = jnp.ones((BATCH,), jnp.float32)

    h = jnp.stack([x[:, :HALF], x[:, HALF:]])
    cnt0 = None
    layers = ((Wl0, Wr0, b0), (Wl1, Wr1, b1), (Wl2, Wr2, b2))
    for i, (Wl, Wr, b) in enumerate(layers):
        right = tc_right(h, Wr, b.reshape(1, D))
        if i == 0:
            summed, cnt = agg0(h, edges_r, zrow, zcnt, ones)
            cnt0 = cnt[0].reshape(N, 1)
        else:
            (summed,) = agg(h, edges_r, zrow, zcnt, ones)
        if i < 2:
            h = comb_mid(summed, cnt0, right, Wl)
        else:
            h = comb_fin(summed, cnt0, right, Wl)
    return h
